# Initial kernel scaffold; baseline (speedup 1.0000x reference)
import jax
import jax.numpy as jnp
from jax.experimental import pallas as pl


def kernel(x_src_0, x_src_1, x_src_2, x_dst_0, x_dst_1, x_dst_2, x_neg_0, x_neg_1, x_neg_2, W0_self, W0_neigh, W1_self, W1_neigh):
    raise NotImplementedError("write your pallas kernel here")


# speedup vs baseline: 1.2721x; 1.2721x over previous
import functools

import jax
import jax.numpy as jnp
from jax.experimental import pallas as pl

B = 512
N0, N1 = 20, 10
F = 128
H0, H1 = 64, 32

GRID = 16
C0 = B // GRID
C1 = C0 * N0


def _body(x0s, x1s, x2s, x0d, x1d, x2d, x0n, x1n, x2n,
          w0s_r, w0n_r, w1s_r, w1n_r, os_r, od_r, on_r):
    W0s = w0s_r[...]
    W0n = w0n_r[...]
    W1s = w1s_r[...]
    W1n = w1n_r[...]
    for x0_r, x1_r, x2_r, o_r in ((x0s, x1s, x2s, os_r),
                                  (x0d, x1d, x2d, od_r),
                                  (x0n, x1n, x2n, on_r)):
        x1b = x1_r[...]
        x2b = x2_r[...]
        x1f = x1b.reshape(C1, F)
        a2 = jnp.mean(x2b, axis=1)
        h1 = jnp.maximum(
            jnp.dot(x1f, W0s, preferred_element_type=jnp.float32)
            + jnp.dot(a2, W0n, preferred_element_type=jnp.float32), 0.0)
        a1 = jnp.mean(x1b, axis=1)
        h0 = jnp.maximum(
            jnp.dot(x0_r[...], W0s, preferred_element_type=jnp.float32)
            + jnp.dot(a1, W0n, preferred_element_type=jnp.float32), 0.0)
        ah1 = jnp.mean(h1.reshape(C0, N0, H0), axis=1)
        o_r[...] = jnp.maximum(
            jnp.dot(h0, W1s, preferred_element_type=jnp.float32)
            + jnp.dot(ah1, W1n, preferred_element_type=jnp.float32), 0.0)


@jax.jit
def kernel(x_src_0, x_src_1, x_src_2, x_dst_0, x_dst_1, x_dst_2,
           x_neg_0, x_neg_1, x_neg_2, W0_self, W0_neigh, W1_self, W1_neigh):
    x1_specs = pl.BlockSpec((C0, N0, F), lambda i: (i, 0, 0))
    x2_specs = pl.BlockSpec((C1, N1, F), lambda i: (i, 0, 0))
    x0_specs = pl.BlockSpec((C0, F), lambda i: (i, 0))
    out_spec = pl.BlockSpec((C0, H1), lambda i: (i, 0))

    def r1(x):
        return x.reshape(B, N0, F)

    def r2(x):
        return x.reshape(B * N0, N1, F)

    in_specs = [x0_specs, x1_specs, x2_specs] * 3 + [
        pl.BlockSpec((F, H0), lambda i: (0, 0)),
        pl.BlockSpec((F, H0), lambda i: (0, 0)),
        pl.BlockSpec((H0, H1), lambda i: (0, 0)),
        pl.BlockSpec((H0, H1), lambda i: (0, 0)),
    ]
    out_shape = [jax.ShapeDtypeStruct((B, H1), jnp.float32)] * 3
    out_specs = [out_spec] * 3

    return tuple(pl.pallas_call(
        _body,
        grid=(GRID,),
        in_specs=in_specs,
        out_specs=out_specs,
        out_shape=out_shape,
    )(x_src_0, r1(x_src_1), r2(x_src_2),
      x_dst_0, r1(x_dst_1), r2(x_dst_2),
      x_neg_0, r1(x_neg_1), r2(x_neg_2),
      W0_self, W0_neigh, W1_self, W1_neigh))


# speedup vs baseline: 3.3465x; 2.6306x over previous
import functools

import jax
import jax.numpy as jnp
from jax.experimental import pallas as pl

B = 512
N0, N1 = 20, 10
F = 128
H0, H1 = 64, 32

GRID = 16
C0 = B // GRID
C1 = C0 * N0


def _body(x0s, x1s, x2s, x0d, x1d, x2d, x0n, x1n, x2n,
          w0s_r, w0n_r, w1s_r, w1n_r, os_r, od_r, on_r):
    W0s = w0s_r[...]
    W0n = w0n_r[...]
    W1s = w1s_r[...]
    W1n = w1n_r[...]
    for x0_r, x1_r, x2_r, o_r in ((x0s, x1s, x2s, os_r),
                                  (x0d, x1d, x2d, od_r),
                                  (x0n, x1n, x2n, on_r)):
        x1f = x1_r[...]
        x1b = x1f.reshape(C0, N0, F)
        x2b = x2_r[...].reshape(C1, N1, F)
        a2 = jnp.mean(x2b, axis=1)
        h1 = jnp.maximum(
            jnp.dot(x1f, W0s, preferred_element_type=jnp.float32)
            + jnp.dot(a2, W0n, preferred_element_type=jnp.float32), 0.0)
        a1 = jnp.mean(x1b, axis=1)
        h0 = jnp.maximum(
            jnp.dot(x0_r[...], W0s, preferred_element_type=jnp.float32)
            + jnp.dot(a1, W0n, preferred_element_type=jnp.float32), 0.0)
        ah1 = jnp.mean(h1.reshape(C0, N0, H0), axis=1)
        o_r[...] = jnp.maximum(
            jnp.dot(h0, W1s, preferred_element_type=jnp.float32)
            + jnp.dot(ah1, W1n, preferred_element_type=jnp.float32), 0.0)


@jax.jit
def kernel(x_src_0, x_src_1, x_src_2, x_dst_0, x_dst_1, x_dst_2,
           x_neg_0, x_neg_1, x_neg_2, W0_self, W0_neigh, W1_self, W1_neigh):
    x1_specs = pl.BlockSpec((C1, F), lambda i: (i, 0))
    x2_specs = pl.BlockSpec((C1 * N1, F), lambda i: (i, 0))
    x0_specs = pl.BlockSpec((C0, F), lambda i: (i, 0))
    out_spec = pl.BlockSpec((C0, H1), lambda i: (i, 0))

    def r1(x):
        return x

    def r2(x):
        return x

    in_specs = [x0_specs, x1_specs, x2_specs] * 3 + [
        pl.BlockSpec((F, H0), lambda i: (0, 0)),
        pl.BlockSpec((F, H0), lambda i: (0, 0)),
        pl.BlockSpec((H0, H1), lambda i: (0, 0)),
        pl.BlockSpec((H0, H1), lambda i: (0, 0)),
    ]
    out_shape = [jax.ShapeDtypeStruct((B, H1), jnp.float32)] * 3
    out_specs = [out_spec] * 3

    return tuple(pl.pallas_call(
        _body,
        grid=(GRID,),
        in_specs=in_specs,
        out_specs=out_specs,
        out_shape=out_shape,
    )(x_src_0, r1(x_src_1), r2(x_src_2),
      x_dst_0, r1(x_dst_1), r2(x_dst_2),
      x_neg_0, r1(x_neg_1), r2(x_neg_2),
      W0_self, W0_neigh, W1_self, W1_neigh))


# speedup vs baseline: 5.5912x; 1.6708x over previous
import functools

import jax
import jax.numpy as jnp
from jax.experimental import pallas as pl

B = 512
N0, N1 = 20, 10
F = 128
H0, H1 = 64, 32

GRID = 16
C0 = B // GRID
C1 = C0 * N0


def _seg_mean(x, n, inner):
    R, Fdim = x.shape
    b = R // inner
    g = inner // n
    X3 = x.reshape(b, inner, Fdim)
    r_ids = jax.lax.broadcasted_iota(jnp.int32, (b, g, inner), 2)
    s_ids = jax.lax.broadcasted_iota(jnp.int32, (b, g, inner), 1)
    S = jnp.where(r_ids // n == s_ids, 1.0, 0.0).astype(x.dtype)
    out = jax.lax.dot_general(
        S, X3, (((2,), (1,)), ((0,), (0,))),
        preferred_element_type=jnp.float32)
    return out.reshape(R // n, Fdim) * (1.0 / n)


def _body(x0s, x1s, x2s, x0d, x1d, x2d, x0n, x1n, x2n,
          w0s_r, w0n_r, w1s_r, w1n_r, os_r, od_r, on_r):
    W0s = w0s_r[...]
    W0n = w0n_r[...]
    W1s = w1s_r[...]
    W1n = w1n_r[...]
    for x0_r, x1_r, x2_r, o_r in ((x0s, x1s, x2s, os_r),
                                  (x0d, x1d, x2d, od_r),
                                  (x0n, x1n, x2n, on_r)):
        x1f = x1_r[...]
        a2 = _seg_mean(x2_r[...], N1, 640)
        h1 = jnp.maximum(
            jnp.dot(x1f, W0s, preferred_element_type=jnp.float32)
            + jnp.dot(a2, W0n, preferred_element_type=jnp.float32), 0.0)
        a1 = _seg_mean(x1f, N0, 160)
        h0 = jnp.maximum(
            jnp.dot(x0_r[...], W0s, preferred_element_type=jnp.float32)
            + jnp.dot(a1, W0n, preferred_element_type=jnp.float32), 0.0)
        ah1 = _seg_mean(h1, N0, 160)
        o_r[...] = jnp.maximum(
            jnp.dot(h0, W1s, preferred_element_type=jnp.float32)
            + jnp.dot(ah1, W1n, preferred_element_type=jnp.float32), 0.0)


@jax.jit
def kernel(x_src_0, x_src_1, x_src_2, x_dst_0, x_dst_1, x_dst_2,
           x_neg_0, x_neg_1, x_neg_2, W0_self, W0_neigh, W1_self, W1_neigh):
    x1_specs = pl.BlockSpec((C1, F), lambda i: (i, 0))
    x2_specs = pl.BlockSpec((C1 * N1, F), lambda i: (i, 0))
    x0_specs = pl.BlockSpec((C0, F), lambda i: (i, 0))
    out_spec = pl.BlockSpec((C0, H1), lambda i: (i, 0))

    def r1(x):
        return x

    def r2(x):
        return x

    in_specs = [x0_specs, x1_specs, x2_specs] * 3 + [
        pl.BlockSpec((F, H0), lambda i: (0, 0)),
        pl.BlockSpec((F, H0), lambda i: (0, 0)),
        pl.BlockSpec((H0, H1), lambda i: (0, 0)),
        pl.BlockSpec((H0, H1), lambda i: (0, 0)),
    ]
    out_shape = [jax.ShapeDtypeStruct((B, H1), jnp.float32)] * 3
    out_specs = [out_spec] * 3

    return tuple(pl.pallas_call(
        _body,
        grid=(GRID,),
        in_specs=in_specs,
        out_specs=out_specs,
        out_shape=out_shape,
    )(x_src_0, r1(x_src_1), r2(x_src_2),
      x_dst_0, r1(x_dst_1), r2(x_dst_2),
      x_neg_0, r1(x_neg_1), r2(x_neg_2),
      W0_self, W0_neigh, W1_self, W1_neigh))


# speedup vs baseline: 5.7998x; 1.0373x over previous
import functools

import jax
import jax.numpy as jnp
from jax.experimental import pallas as pl

B = 512
N0, N1 = 20, 10
F = 128
H0, H1 = 64, 32

GRID = 8
C0 = B // GRID
C1 = C0 * N0


def _seg_mean(x, n, inner):
    R, Fdim = x.shape
    b = R // inner
    g = inner // n
    X3 = x.reshape(b, inner, Fdim)
    r_ids = jax.lax.broadcasted_iota(jnp.int32, (b, g, inner), 2)
    s_ids = jax.lax.broadcasted_iota(jnp.int32, (b, g, inner), 1)
    S = jnp.where(r_ids // n == s_ids, 1.0, 0.0).astype(x.dtype)
    out = jax.lax.dot_general(
        S, X3, (((2,), (1,)), ((0,), (0,))),
        preferred_element_type=jnp.float32)
    return out.reshape(R // n, Fdim) * (1.0 / n)


def _body(x0s, x1s, x2s, x0d, x1d, x2d, x0n, x1n, x2n,
          w0s_r, w0n_r, w1s_r, w1n_r, os_r, od_r, on_r):
    W0s = w0s_r[...]
    W0n = w0n_r[...]
    W1s = w1s_r[...]
    W1n = w1n_r[...]
    for x0_r, x1_r, x2_r, o_r in ((x0s, x1s, x2s, os_r),
                                  (x0d, x1d, x2d, od_r),
                                  (x0n, x1n, x2n, on_r)):
        x1f = x1_r[...]
        a2 = _seg_mean(x2_r[...], N1, 640)
        h1 = jnp.maximum(
            jnp.dot(x1f, W0s, preferred_element_type=jnp.float32)
            + jnp.dot(a2, W0n, preferred_element_type=jnp.float32), 0.0)
        a1 = _seg_mean(x1f, N0, 160)
        h0 = jnp.maximum(
            jnp.dot(x0_r[...], W0s, preferred_element_type=jnp.float32)
            + jnp.dot(a1, W0n, preferred_element_type=jnp.float32), 0.0)
        ah1 = _seg_mean(h1, N0, 160)
        o_r[...] = jnp.maximum(
            jnp.dot(h0, W1s, preferred_element_type=jnp.float32)
            + jnp.dot(ah1, W1n, preferred_element_type=jnp.float32), 0.0)


@jax.jit
def kernel(x_src_0, x_src_1, x_src_2, x_dst_0, x_dst_1, x_dst_2,
           x_neg_0, x_neg_1, x_neg_2, W0_self, W0_neigh, W1_self, W1_neigh):
    x1_specs = pl.BlockSpec((C1, F), lambda i: (i, 0))
    x2_specs = pl.BlockSpec((C1 * N1, F), lambda i: (i, 0))
    x0_specs = pl.BlockSpec((C0, F), lambda i: (i, 0))
    out_spec = pl.BlockSpec((C0, H1), lambda i: (i, 0))

    def r1(x):
        return x

    def r2(x):
        return x

    in_specs = [x0_specs, x1_specs, x2_specs] * 3 + [
        pl.BlockSpec((F, H0), lambda i: (0, 0)),
        pl.BlockSpec((F, H0), lambda i: (0, 0)),
        pl.BlockSpec((H0, H1), lambda i: (0, 0)),
        pl.BlockSpec((H0, H1), lambda i: (0, 0)),
    ]
    out_shape = [jax.ShapeDtypeStruct((B, H1), jnp.float32)] * 3
    out_specs = [out_spec] * 3

    return tuple(pl.pallas_call(
        _body,
        grid=(GRID,),
        in_specs=in_specs,
        out_specs=out_specs,
        out_shape=out_shape,
    )(x_src_0, r1(x_src_1), r2(x_src_2),
      x_dst_0, r1(x_dst_1), r2(x_dst_2),
      x_neg_0, r1(x_neg_1), r2(x_neg_2),
      W0_self, W0_neigh, W1_self, W1_neigh))
